# baseline (speedup 1.0000x reference)
import functools

import jax
import jax.numpy as jnp
from jax import lax
from jax.experimental import pallas as pl
from jax.experimental.pallas import tpu as pltpu
from jax.experimental.pallas import tpu_sc as plsc

_N = 32 * 512 * 512

_INFO = plsc.get_sparse_core_info()
_NC = _INFO.num_cores
_NS = _INFO.num_subcores
_L = _INFO.num_lanes
_NW = _NC * _NS
_PER_W = _N // _NW
_CHUNK = 16384
_NCHUNK = _PER_W // _CHUNK
_VECS = _CHUNK // _L


def _make_sc_kernel():
    mesh = plsc.VectorSubcoreMesh(core_axis_name="c", subcore_axis_name="s")

    @functools.partial(
        pl.kernel,
        mesh=mesh,
        out_type=jax.ShapeDtypeStruct((_NW, 2, _L), jnp.float32),
        scratch_types=[
            pltpu.VMEM((_CHUNK,), jnp.float32),
            pltpu.VMEM((_CHUNK,), jnp.float32),
            pltpu.VMEM((_CHUNK,), jnp.float32),
            pltpu.VMEM((_CHUNK,), jnp.float32),
            pltpu.VMEM((_CHUNK,), jnp.float32),
            pltpu.VMEM((_CHUNK,), jnp.float32),
            pltpu.VMEM((2, _L), jnp.float32),
            pltpu.SemaphoreType.DMA,
            pltpu.SemaphoreType.DMA,
        ],
    )
    def masked_gauss(m_hbm, lv_hbm, t_hbm, out_hbm,
                     m0, m1, lv0, lv1, t0, t1, acc, sem0, sem1):
        wid = lax.axis_index("s") * _NC + lax.axis_index("c")
        base = wid * _PER_W
        bufs = ((m0, lv0, t0, sem0), (m1, lv1, t1, sem1))

        def start(g, slot):
            mb, lvb, tb, sem = bufs[slot]
            off = base + g * _CHUNK
            return (
                pltpu.async_copy(m_hbm.at[pl.ds(off, _CHUNK)], mb, sem),
                pltpu.async_copy(lv_hbm.at[pl.ds(off, _CHUNK)], lvb, sem),
                pltpu.async_copy(t_hbm.at[pl.ds(off, _CHUNK)], tb, sem),
            )

        zero = jnp.zeros((_L,), jnp.float32)
        carry = (zero, zero)
        inflight = start(0, 0)
        for g in range(_NCHUNK):
            slot = g % 2
            nxt = start(g + 1, 1 - slot) if g + 1 < _NCHUNK else None
            for h in inflight:
                h.wait()
            mb, lvb, tb, _ = bufs[slot]

            def body(i, c, mb=mb, lvb=lvb, tb=tb):
                s, n = c
                sl = pl.ds(i * _L, _L)
                m = mb[sl]
                lv = lvb[sl]
                t = tb[sl]
                msk = t > 0.0
                d = t - m
                e = jnp.exp(-lv) * (d * d) + lv
                return (s + jnp.where(msk, e, 0.0),
                        n + jnp.where(msk, 1.0, 0.0))

            carry = lax.fori_loop(0, _VECS, body, carry)
            inflight = nxt
        s, n = carry
        acc[0, :] = s
        acc[1, :] = n
        pltpu.sync_copy(acc, out_hbm.at[wid])

    return masked_gauss


_sc_call = _make_sc_kernel()


@jax.jit
def kernel(means, log_vars, targets):
    m = means.reshape(_N)
    lv = log_vars.reshape(_N)
    t = targets.reshape(_N)
    parts = _sc_call(m, lv, t)
    return parts[:, 0, :].sum() / parts[:, 1, :].sum()


# speedup vs baseline: 1.1523x; 1.1523x over previous
import functools

import jax
import jax.numpy as jnp
from jax import lax
from jax.experimental import pallas as pl
from jax.experimental.pallas import tpu as pltpu
from jax.experimental.pallas import tpu_sc as plsc

_N = 32 * 512 * 512

_INFO = plsc.get_sparse_core_info()
_NC = _INFO.num_cores
_NS = _INFO.num_subcores
_L = _INFO.num_lanes
_NW = _NC * _NS
_PER_W = _N // _NW
_CHUNK = 16384
_NCHUNK = _PER_W // _CHUNK
_VECS = _CHUNK // _L


def _make_sc_kernel():
    mesh = plsc.VectorSubcoreMesh(core_axis_name="c", subcore_axis_name="s")

    @functools.partial(
        pl.kernel,
        mesh=mesh,
        out_type=jax.ShapeDtypeStruct((_NW, 2, _L), jnp.float32),
        scratch_types=[
            pltpu.VMEM((_CHUNK,), jnp.float32),
            pltpu.VMEM((_CHUNK,), jnp.float32),
            pltpu.VMEM((_CHUNK,), jnp.float32),
            pltpu.VMEM((_CHUNK,), jnp.float32),
            pltpu.VMEM((_CHUNK,), jnp.float32),
            pltpu.VMEM((_CHUNK,), jnp.float32),
            pltpu.VMEM((2, _L), jnp.float32),
            pltpu.SemaphoreType.DMA,
            pltpu.SemaphoreType.DMA,
        ],
    )
    def masked_gauss(m_hbm, lv_hbm, t_hbm, out_hbm,
                     m0, m1, lv0, lv1, t0, t1, acc, sem0, sem1):
        wid = lax.axis_index("s") * _NC + lax.axis_index("c")
        base = wid * _PER_W
        bufs = ((m0, lv0, t0, sem0), (m1, lv1, t1, sem1))

        def start(g, slot):
            mb, lvb, tb, sem = bufs[slot]
            off = base + g * _CHUNK
            return (
                pltpu.async_copy(m_hbm.at[pl.ds(off, _CHUNK)], mb, sem),
                pltpu.async_copy(lv_hbm.at[pl.ds(off, _CHUNK)], lvb, sem),
                pltpu.async_copy(t_hbm.at[pl.ds(off, _CHUNK)], tb, sem),
            )

        zero = jnp.zeros((_L,), jnp.float32)
        carry = (zero, zero)
        inflight = start(0, 0)
        for g in range(_NCHUNK):
            slot = g % 2
            nxt = start(g + 1, 1 - slot) if g + 1 < _NCHUNK else None
            for h in inflight:
                h.wait()
            mb, lvb, tb, _ = bufs[slot]

            def body(i, c, mb=mb, lvb=lvb, tb=tb):
                s, n = c
                sl = pl.ds(i * _L, _L)
                m = mb[sl]
                lv = lvb[sl]
                t = tb[sl]
                msk = t > 0.0
                d = t - m
                e = jnp.exp(-lv) * (d * d) + lv
                return (s + jnp.where(msk, e, 0.0),
                        n + jnp.where(msk, 1.0, 0.0))

            carry = lax.fori_loop(0, _VECS, body, carry, unroll=8)
            inflight = nxt
        s, n = carry
        acc[0, :] = s
        acc[1, :] = n
        pltpu.sync_copy(acc, out_hbm.at[wid])

    return masked_gauss


_sc_call = _make_sc_kernel()


@jax.jit
def kernel(means, log_vars, targets):
    m = means.reshape(_N)
    lv = log_vars.reshape(_N)
    t = targets.reshape(_N)
    parts = _sc_call(m, lv, t)
    return parts[:, 0, :].sum() / parts[:, 1, :].sum()


# speedup vs baseline: 2.5475x; 2.2108x over previous
import functools

import jax
import jax.numpy as jnp
from jax import lax
from jax.experimental import pallas as pl
from jax.experimental.pallas import tpu as pltpu
from jax.experimental.pallas import tpu_sc as plsc

_B = 32
_H = 512
_W = 512

_INFO = plsc.get_sparse_core_info()
_NC = _INFO.num_cores
_NS = _INFO.num_subcores
_L = _INFO.num_lanes
_NW = _NC * _NS

_ROWS = 32
_NCHUNK = _H // _ROWS
_VECS = _ROWS * _W // _L
_RVECS = _W // _L


def _make_sc_kernel():
    mesh = plsc.VectorSubcoreMesh(core_axis_name="c", subcore_axis_name="s")

    @functools.partial(
        pl.kernel,
        mesh=mesh,
        out_type=jax.ShapeDtypeStruct((_NW, 2, _L), jnp.float32),
        scratch_types=[
            pltpu.VMEM((_ROWS, _W), jnp.float32),
            pltpu.VMEM((_ROWS, _W), jnp.float32),
            pltpu.VMEM((_ROWS, _W), jnp.float32),
            pltpu.VMEM((_ROWS, _W), jnp.float32),
            pltpu.VMEM((_ROWS, _W), jnp.float32),
            pltpu.VMEM((_ROWS, _W), jnp.float32),
            pltpu.VMEM((2, _L), jnp.float32),
            pltpu.SemaphoreType.DMA,
            pltpu.SemaphoreType.DMA,
        ],
    )
    def masked_gauss(m_hbm, lv_hbm, t_hbm, out_hbm,
                     m0, m1, lv0, lv1, t0, t1, acc, sem0, sem1):
        wid = lax.axis_index("s") * _NC + lax.axis_index("c")
        bufs = ((m0, lv0, t0, sem0), (m1, lv1, t1, sem1))

        def start(g, slot):
            mb, lvb, tb, sem = bufs[slot]
            rows = pl.ds(g * _ROWS, _ROWS)
            return (
                pltpu.async_copy(m_hbm.at[wid, 0, rows], mb, sem),
                pltpu.async_copy(lv_hbm.at[wid, 0, rows], lvb, sem),
                pltpu.async_copy(t_hbm.at[wid, rows], tb, sem),
            )

        zero = jnp.zeros((_L,), jnp.float32)
        carry = (zero, zero)
        inflight = start(0, 0)
        for g in range(_NCHUNK):
            slot = g % 2
            nxt = start(g + 1, 1 - slot) if g + 1 < _NCHUNK else None
            for h in inflight:
                h.wait()
            mb, lvb, tb, _ = bufs[slot]

            def body(i, c, mb=mb, lvb=lvb, tb=tb):
                s, n = c
                r = lax.shift_right_logical(i, 5)
                col = pl.multiple_of(
                    lax.shift_left(lax.bitwise_and(i, _RVECS - 1), 4), _L)
                sl = pl.ds(col, _L)
                m = mb[r, sl]
                lv = lvb[r, sl]
                t = tb[r, sl]
                msk = t > 0.0
                d = t - m
                e = jnp.exp(-lv) * (d * d) + lv
                return (s + jnp.where(msk, e, 0.0),
                        n + jnp.where(msk, 1.0, 0.0))

            carry = lax.fori_loop(0, _VECS, body, carry, unroll=8)
            inflight = nxt
        s, n = carry
        acc[0, :] = s
        acc[1, :] = n
        pltpu.sync_copy(acc, out_hbm.at[wid])

    return masked_gauss


_sc_call = _make_sc_kernel()


@jax.jit
def kernel(means, log_vars, targets):
    parts = _sc_call(means, log_vars, targets)
    return parts[:, 0, :].sum() / parts[:, 1, :].sum()
